# baseline (speedup 1.0000x reference)
import functools
import math

import jax
import jax.numpy as jnp
from jax import lax
from jax.experimental import pallas as pl
from jax.experimental.pallas import tpu as pltpu
from jax.experimental.pallas import tpu_sc as plsc

NUM_CODES = 8192
CODE_DIM = 32
ROW_BLOCK = 512


def _vq_kernel(z2_blk, embed_t, cs, codes_out, scal_out,
               active_out, soft_acc, esq, msum, *, n_rows, n_blocks):
    i = pl.program_id(0)

    @pl.when(i == 0)
    def _init():
        esq[...] = jnp.sum(embed_t[...] * embed_t[...], axis=0, keepdims=True)
        soft_acc[...] = jnp.zeros_like(soft_acc)
        msum[0, 0] = 0.0

    z2 = z2_blk[...]
    zsq = 0.25 * jnp.sum(z2 * z2, axis=1, keepdims=True)
    prod2 = jnp.dot(z2, embed_t[...], preferred_element_type=jnp.float32)
    dist = (zsq - prod2) + esq[...]

    rowmin = jnp.min(dist, axis=1, keepdims=True)
    lane_f = jax.lax.broadcasted_iota(
        jnp.int32, (1, dist.shape[1]), 1).astype(jnp.float32)
    codes_f = jnp.min(jnp.where(dist == rowmin, lane_f, jnp.float32(NUM_CODES)),
                      axis=1, keepdims=True)
    codes = codes_f.astype(jnp.int32)

    p = jnp.exp(rowmin - dist)
    ones_k = jnp.full((dist.shape[1], 1), 1.0, jnp.float32)
    zsum = jnp.dot(p, ones_k, preferred_element_type=jnp.float32)
    invz_row = jnp.transpose(1.0 / zsum)
    soft_acc[...] += jnp.dot(invz_row, p, preferred_element_type=jnp.float32)

    codes_out[...] = codes
    msum[0, 0] += jnp.sum(rowmin)

    @pl.when(i == n_blocks - 1)
    def _finish():
        n = jnp.float32(n_rows)
        avg_soft = soft_acc[...] / n
        usage_entropy = -jnp.sum(avg_soft * jnp.log(avg_soft + 1e-10))
        diversity_loss = -usage_entropy / math.log(NUM_CODES)
        loss = msum[0, 0] / (n * CODE_DIM)
        lane8 = jax.lax.broadcasted_iota(jnp.int32, (1, 8), 1)
        scal_out[...] = (loss * (lane8 <= 1)
                         + diversity_loss * (lane8 == 2)
                         + usage_entropy * (lane8 == 3))
        active = jnp.sum((cs[...] > 1.0).astype(jnp.int32))
        active_out[...] = jnp.broadcast_to(active, (1, 1))


def _tc_stats(flat_z, embed, cluster_size):
    n_rows = flat_z.shape[0]
    n_blocks = n_rows // ROW_BLOCK
    embed_t = embed.T
    cs = cluster_size.reshape(1, NUM_CODES)

    out_shapes = (
        jax.ShapeDtypeStruct((n_rows, 1), jnp.int32),
        jax.ShapeDtypeStruct((1, 8), jnp.float32),
        jax.ShapeDtypeStruct((1, 1), jnp.int32),
    )
    return pl.pallas_call(
        functools.partial(_vq_kernel, n_rows=n_rows, n_blocks=n_blocks),
        grid=(n_blocks,),
        in_specs=[
            pl.BlockSpec((ROW_BLOCK, CODE_DIM), lambda i: (i, 0)),
            pl.BlockSpec((CODE_DIM, NUM_CODES), lambda i: (0, 0)),
            pl.BlockSpec((1, NUM_CODES), lambda i: (0, 0)),
        ],
        out_specs=(
            pl.BlockSpec((ROW_BLOCK, 1), lambda i: (i, 0)),
            pl.BlockSpec((1, 8), lambda i: (0, 0)),
            pl.BlockSpec((1, 1), lambda i: (0, 0)),
        ),
        out_shape=out_shapes,
        scratch_shapes=[
            pltpu.VMEM((1, NUM_CODES), jnp.float32),
            pltpu.VMEM((1, NUM_CODES), jnp.float32),
            pltpu.SMEM((1, 1), jnp.float32),
        ],
    )(2.0 * flat_z, embed_t, cs)


def _sc_gather_hist(embed, codes_flat, zeros_k, ones_b):
    info = plsc.get_sparse_core_info()
    nw = info.num_cores * info.num_subcores
    n = codes_flat.shape[0]
    b_per_w = n // nw
    mesh = plsc.VectorSubcoreMesh(core_axis_name="c", subcore_axis_name="s")

    @functools.partial(
        pl.kernel,
        mesh=mesh,
        out_type=(
            jax.ShapeDtypeStruct((n, CODE_DIM), jnp.float32),
            jax.ShapeDtypeStruct((info.num_cores, NUM_CODES), jnp.float32),
        ),
        scratch_types=[
            pltpu.VMEM((b_per_w,), jnp.int32),
            pltpu.VMEM((b_per_w, CODE_DIM), jnp.float32),
            pltpu.VMEM((b_per_w,), jnp.float32),
            pltpu.VMEM_SHARED((NUM_CODES,), jnp.float32),
            pltpu.SemaphoreType.DMA,
        ],
        compiler_params=pltpu.CompilerParams(use_tc_tiling_on_sc=False),
    )
    def gather_k(table_hbm, idx_hbm, zeros_hbm, ones_hbm,
                 out_hbm, cnt_hbm, idx_v, rows_v, ones_v, cnt_sh, sem):
        cid = lax.axis_index("c")
        sid = lax.axis_index("s")
        wid = sid * info.num_cores + cid
        base = wid * b_per_w

        @pl.when(sid == 0)
        def _zero():
            pltpu.sync_copy(zeros_hbm, cnt_sh)
        pltpu.sync_copy(idx_hbm.at[pl.ds(base, b_per_w)], idx_v)
        pltpu.sync_copy(ones_hbm.at[pl.ds(0, b_per_w)], ones_v)
        pltpu.async_copy(table_hbm.at[idx_v], rows_v, sem).wait()
        pltpu.sync_copy(rows_v, out_hbm.at[pl.ds(base, b_per_w)])
        plsc.subcore_barrier()
        pltpu.sync_copy(ones_v, cnt_sh.at[idx_v], add=True)
        plsc.subcore_barrier()

        @pl.when(sid == 0)
        def _flush():
            pltpu.sync_copy(cnt_sh, cnt_hbm.at[cid])

    return gather_k(embed, codes_flat, zeros_k, ones_b)


def _perp_kernel(cnt, scal_in, scal_out, *, n_rows):
    hard_avg = (cnt[0:1, :] + cnt[1:2, :]) / jnp.float32(n_rows)
    perp = jnp.exp(-jnp.sum(hard_avg * jnp.log(hard_avg + 1e-10)))
    lane8 = jax.lax.broadcasted_iota(jnp.int32, (1, 8), 1)
    scal_out[...] = jnp.where(lane8 == 4, perp, scal_in[...])


def _tc_perplexity(cnt_partials, scal, n_rows):
    return pl.pallas_call(
        functools.partial(_perp_kernel, n_rows=n_rows),
        out_shape=jax.ShapeDtypeStruct((1, 8), jnp.float32),
    )(cnt_partials, scal)


@jax.jit
def kernel(z, embed, cluster_size):
    orig_shape = z.shape
    flat_z = z.reshape(-1, CODE_DIM)
    n_rows = flat_z.shape[0]
    codes, scal, active = _tc_stats(flat_z, embed, cluster_size)
    codes_flat = codes.reshape(-1)
    zeros_k = jnp.zeros((NUM_CODES,), jnp.float32)
    ones_b = jnp.ones((n_rows,), jnp.float32)
    zq, cnt_partials = _sc_gather_hist(embed, codes_flat, zeros_k, ones_b)
    scal = _tc_perplexity(cnt_partials, scal, n_rows)

    z_q_out = zq.reshape(orig_shape)
    codes_out = codes.reshape(orig_shape[:-1])
    return (z_q_out, codes_out,
            scal[0, 0], scal[0, 1], scal[0, 2], scal[0, 3], scal[0, 4],
            active[0, 0])


# speedup vs baseline: 1.0170x; 1.0170x over previous
import functools
import math

import jax
import jax.numpy as jnp
from jax import lax
from jax.experimental import pallas as pl
from jax.experimental.pallas import tpu as pltpu
from jax.experimental.pallas import tpu_sc as plsc

NUM_CODES = 8192
CODE_DIM = 32
ROW_BLOCK = 512


def _vq_kernel(z_blk, embed_t, cs, codes_out, scal_out,
               active_out, soft_acc, esq, msum, *, n_rows, n_blocks):
    i = pl.program_id(0)

    @pl.when(i == 0)
    def _init():
        esq[...] = jnp.sum(embed_t[...] * embed_t[...], axis=0, keepdims=True)
        soft_acc[...] = jnp.zeros_like(soft_acc)
        msum[0, 0] = 0.0

    z2 = 2.0 * z_blk[...]
    zsq = 0.25 * jnp.sum(z2 * z2, axis=1, keepdims=True)
    prod2 = jnp.dot(z2, embed_t[...], preferred_element_type=jnp.float32)
    dist = (zsq - prod2) + esq[...]

    rowmin = jnp.min(dist, axis=1, keepdims=True)
    lane = jax.lax.broadcasted_iota(jnp.int32, (1, dist.shape[1]), 1)
    codes = jnp.min(jnp.where(dist == rowmin, lane, NUM_CODES),
                    axis=1, keepdims=True)

    p = jnp.exp(rowmin - dist)
    ones_k = jnp.full((dist.shape[1], 1), 1.0, jnp.float32)
    zsum = jnp.dot(p, ones_k, preferred_element_type=jnp.float32)
    invz_row = jnp.transpose(1.0 / zsum)
    soft_acc[...] += jnp.dot(invz_row, p, preferred_element_type=jnp.float32)

    codes_out[...] = codes
    msum[0, 0] += jnp.sum(rowmin)

    @pl.when(i == n_blocks - 1)
    def _finish():
        n = jnp.float32(n_rows)
        avg_soft = soft_acc[...] / n
        usage_entropy = -jnp.sum(avg_soft * jnp.log(avg_soft + 1e-10))
        diversity_loss = -usage_entropy / math.log(NUM_CODES)
        loss = msum[0, 0] / (n * CODE_DIM)
        lane8 = jax.lax.broadcasted_iota(jnp.int32, (1, 8), 1)
        scal_out[...] = (loss * (lane8 <= 1)
                         + diversity_loss * (lane8 == 2)
                         + usage_entropy * (lane8 == 3))
        active = jnp.sum((cs[...] > 1.0).astype(jnp.int32))
        active_out[...] = jnp.broadcast_to(active, (1, 1))


def _tc_stats(flat_z, embed, cluster_size):
    n_rows = flat_z.shape[0]
    n_blocks = n_rows // ROW_BLOCK
    embed_t = embed.T
    cs = cluster_size.reshape(1, NUM_CODES)

    out_shapes = (
        jax.ShapeDtypeStruct((n_rows, 1), jnp.int32),
        jax.ShapeDtypeStruct((1, 8), jnp.float32),
        jax.ShapeDtypeStruct((1, 1), jnp.int32),
    )
    return pl.pallas_call(
        functools.partial(_vq_kernel, n_rows=n_rows, n_blocks=n_blocks),
        grid=(n_blocks,),
        in_specs=[
            pl.BlockSpec((ROW_BLOCK, CODE_DIM), lambda i: (i, 0)),
            pl.BlockSpec((CODE_DIM, NUM_CODES), lambda i: (0, 0)),
            pl.BlockSpec((1, NUM_CODES), lambda i: (0, 0)),
        ],
        out_specs=(
            pl.BlockSpec((ROW_BLOCK, 1), lambda i: (i, 0)),
            pl.BlockSpec((1, 8), lambda i: (0, 0)),
            pl.BlockSpec((1, 1), lambda i: (0, 0)),
        ),
        out_shape=out_shapes,
        scratch_shapes=[
            pltpu.VMEM((1, NUM_CODES), jnp.float32),
            pltpu.VMEM((1, NUM_CODES), jnp.float32),
            pltpu.SMEM((1, 1), jnp.float32),
        ],
    )(flat_z, embed_t, cs)


def _sc_gather_hist(embed, codes_flat, zeros_k, ones_b):
    info = plsc.get_sparse_core_info()
    nw = info.num_cores * info.num_subcores
    n = codes_flat.shape[0]
    b_per_w = n // nw
    mesh = plsc.VectorSubcoreMesh(core_axis_name="c", subcore_axis_name="s")

    @functools.partial(
        pl.kernel,
        mesh=mesh,
        out_type=(
            jax.ShapeDtypeStruct((n, CODE_DIM), jnp.float32),
            jax.ShapeDtypeStruct((info.num_cores, NUM_CODES), jnp.float32),
        ),
        scratch_types=[
            pltpu.VMEM((b_per_w,), jnp.int32),
            pltpu.VMEM((b_per_w, CODE_DIM), jnp.float32),
            pltpu.VMEM((b_per_w,), jnp.float32),
            pltpu.VMEM_SHARED((NUM_CODES,), jnp.float32),
            pltpu.SemaphoreType.DMA,
        ],
        compiler_params=pltpu.CompilerParams(use_tc_tiling_on_sc=False),
    )
    def gather_k(table_hbm, idx_hbm, zeros_hbm, ones_hbm,
                 out_hbm, cnt_hbm, idx_v, rows_v, ones_v, cnt_sh, sem):
        cid = lax.axis_index("c")
        sid = lax.axis_index("s")
        wid = sid * info.num_cores + cid
        base = wid * b_per_w

        @pl.when(sid == 0)
        def _zero():
            pltpu.sync_copy(zeros_hbm, cnt_sh)
        pltpu.sync_copy(idx_hbm.at[pl.ds(base, b_per_w)], idx_v)
        pltpu.sync_copy(ones_hbm.at[pl.ds(0, b_per_w)], ones_v)
        pltpu.async_copy(table_hbm.at[idx_v], rows_v, sem).wait()
        pltpu.sync_copy(rows_v, out_hbm.at[pl.ds(base, b_per_w)])
        plsc.subcore_barrier()
        pltpu.sync_copy(ones_v, cnt_sh.at[idx_v], add=True)
        plsc.subcore_barrier()

        @pl.when(sid == 0)
        def _flush():
            pltpu.sync_copy(cnt_sh, cnt_hbm.at[cid])

    return gather_k(embed, codes_flat, zeros_k, ones_b)


def _perp_kernel(cnt, scal_in, scal_out, *, n_rows):
    hard_avg = (cnt[0:1, :] + cnt[1:2, :]) / jnp.float32(n_rows)
    perp = jnp.exp(-jnp.sum(hard_avg * jnp.log(hard_avg + 1e-10)))
    lane8 = jax.lax.broadcasted_iota(jnp.int32, (1, 8), 1)
    scal_out[...] = jnp.where(lane8 == 4, perp, scal_in[...])


def _tc_perplexity(cnt_partials, scal, n_rows):
    return pl.pallas_call(
        functools.partial(_perp_kernel, n_rows=n_rows),
        out_shape=jax.ShapeDtypeStruct((1, 8), jnp.float32),
    )(cnt_partials, scal)


@jax.jit
def kernel(z, embed, cluster_size):
    orig_shape = z.shape
    flat_z = z.reshape(-1, CODE_DIM)
    n_rows = flat_z.shape[0]
    codes, scal, active = _tc_stats(flat_z, embed, cluster_size)
    codes_flat = codes.reshape(-1)
    zeros_k = jnp.zeros((NUM_CODES,), jnp.float32)
    ones_b = jnp.ones((n_rows,), jnp.float32)
    zq, cnt_partials = _sc_gather_hist(embed, codes_flat, zeros_k, ones_b)
    scal = _tc_perplexity(cnt_partials, scal, n_rows)

    z_q_out = zq.reshape(orig_shape)
    codes_out = codes.reshape(orig_shape[:-1])
    return (z_q_out, codes_out,
            scal[0, 0], scal[0, 1], scal[0, 2], scal[0, 3], scal[0, 4],
            active[0, 0])


# speedup vs baseline: 1.2467x; 1.2259x over previous
import functools
import math

import jax
import jax.numpy as jnp
from jax import lax
from jax.experimental import pallas as pl
from jax.experimental.pallas import tpu as pltpu
from jax.experimental.pallas import tpu_sc as plsc

NUM_CODES = 8192
CODE_DIM = 32
ROW_BLOCK = 512


def _vq_kernel(z_blk, embed_t, cs, codes_out, scal_out,
               active_out, soft_acc, esq, msum, *, n_rows, n_blocks):
    i = pl.program_id(0)

    @pl.when(i == 0)
    def _init():
        esq[...] = jnp.sum(embed_t[...] * embed_t[...], axis=0, keepdims=True)
        soft_acc[...] = jnp.zeros_like(soft_acc)
        msum[0, 0] = 0.0

    z2 = 2.0 * z_blk[...]
    zsq = 0.25 * jnp.sum(z2 * z2, axis=1, keepdims=True)
    prod2 = jnp.dot(z2, embed_t[...], preferred_element_type=jnp.float32)
    dist = (zsq - prod2) + esq[...]

    rowmin = jnp.min(dist, axis=1, keepdims=True)
    lane = jax.lax.broadcasted_iota(jnp.int32, (1, dist.shape[1]), 1)
    codes = jnp.min(jnp.where(dist == rowmin, lane, NUM_CODES),
                    axis=1, keepdims=True)

    p = jnp.exp(rowmin - dist)
    zsum = jnp.sum(p, axis=1, keepdims=True)
    invz_row = jnp.transpose(1.0 / zsum)
    soft_acc[...] += jnp.dot(invz_row, p, preferred_element_type=jnp.float32)

    codes_out[...] = codes
    msum[0, 0] += jnp.sum(rowmin)

    @pl.when(i == n_blocks - 1)
    def _finish():
        n = jnp.float32(n_rows)
        avg_soft = soft_acc[...] / n
        usage_entropy = -jnp.sum(avg_soft * jnp.log(avg_soft + 1e-10))
        diversity_loss = -usage_entropy / math.log(NUM_CODES)
        loss = msum[0, 0] / (n * CODE_DIM)
        lane8 = jax.lax.broadcasted_iota(jnp.int32, (1, 8), 1)
        scal_out[...] = (loss * (lane8 <= 1)
                         + diversity_loss * (lane8 == 2)
                         + usage_entropy * (lane8 == 3))
        active = jnp.sum((cs[...] > 1.0).astype(jnp.int32))
        active_out[...] = jnp.broadcast_to(active, (1, 1))


def _tc_stats(flat_z, embed, cluster_size):
    n_rows = flat_z.shape[0]
    n_blocks = n_rows // ROW_BLOCK
    embed_t = embed.T
    cs = cluster_size.reshape(1, NUM_CODES)

    out_shapes = (
        jax.ShapeDtypeStruct((n_rows, 1), jnp.int32),
        jax.ShapeDtypeStruct((1, 8), jnp.float32),
        jax.ShapeDtypeStruct((1, 1), jnp.int32),
    )
    return pl.pallas_call(
        functools.partial(_vq_kernel, n_rows=n_rows, n_blocks=n_blocks),
        grid=(n_blocks,),
        in_specs=[
            pl.BlockSpec((ROW_BLOCK, CODE_DIM), lambda i: (i, 0)),
            pl.BlockSpec((CODE_DIM, NUM_CODES), lambda i: (0, 0)),
            pl.BlockSpec((1, NUM_CODES), lambda i: (0, 0)),
        ],
        out_specs=(
            pl.BlockSpec((ROW_BLOCK, 1), lambda i: (i, 0)),
            pl.BlockSpec((1, 8), lambda i: (0, 0)),
            pl.BlockSpec((1, 1), lambda i: (0, 0)),
        ),
        out_shape=out_shapes,
        scratch_shapes=[
            pltpu.VMEM((1, NUM_CODES), jnp.float32),
            pltpu.VMEM((1, NUM_CODES), jnp.float32),
            pltpu.SMEM((1, 1), jnp.float32),
        ],
    )(flat_z, embed_t, cs)


def _sc_gather_hist(embed, codes_flat, zeros_k, ones_b):
    info = plsc.get_sparse_core_info()
    nw = info.num_cores * info.num_subcores
    n = codes_flat.shape[0]
    b_per_w = n // nw
    mesh = plsc.VectorSubcoreMesh(core_axis_name="c", subcore_axis_name="s")

    @functools.partial(
        pl.kernel,
        mesh=mesh,
        out_type=(
            jax.ShapeDtypeStruct((n, CODE_DIM), jnp.float32),
            jax.ShapeDtypeStruct((info.num_cores, NUM_CODES), jnp.float32),
        ),
        scratch_types=[
            pltpu.VMEM((b_per_w,), jnp.int32),
            pltpu.VMEM((b_per_w, CODE_DIM), jnp.float32),
            pltpu.VMEM((b_per_w,), jnp.float32),
            pltpu.VMEM_SHARED((NUM_CODES,), jnp.float32),
            pltpu.SemaphoreType.DMA,
        ],
        compiler_params=pltpu.CompilerParams(use_tc_tiling_on_sc=False),
    )
    def gather_k(table_hbm, idx_hbm, zeros_hbm, ones_hbm,
                 out_hbm, cnt_hbm, idx_v, rows_v, ones_v, cnt_sh, sem):
        cid = lax.axis_index("c")
        sid = lax.axis_index("s")
        wid = sid * info.num_cores + cid
        base = wid * b_per_w

        @pl.when(sid == 0)
        def _zero():
            pltpu.sync_copy(zeros_hbm, cnt_sh)
        pltpu.sync_copy(idx_hbm.at[pl.ds(base, b_per_w)], idx_v)
        pltpu.sync_copy(ones_hbm.at[pl.ds(0, b_per_w)], ones_v)
        pltpu.async_copy(table_hbm.at[idx_v], rows_v, sem).wait()
        pltpu.sync_copy(rows_v, out_hbm.at[pl.ds(base, b_per_w)])
        plsc.subcore_barrier()
        pltpu.sync_copy(ones_v, cnt_sh.at[idx_v], add=True)
        plsc.subcore_barrier()

        @pl.when(sid == 0)
        def _flush():
            pltpu.sync_copy(cnt_sh, cnt_hbm.at[cid])

    return gather_k(embed, codes_flat, zeros_k, ones_b)


def _perp_kernel(cnt, scal_in, scal_out, *, n_rows):
    hard_avg = (cnt[0:1, :] + cnt[1:2, :]) / jnp.float32(n_rows)
    perp = jnp.exp(-jnp.sum(hard_avg * jnp.log(hard_avg + 1e-10)))
    lane8 = jax.lax.broadcasted_iota(jnp.int32, (1, 8), 1)
    scal_out[...] = jnp.where(lane8 == 4, perp, scal_in[...])


def _tc_perplexity(cnt_partials, scal, n_rows):
    return pl.pallas_call(
        functools.partial(_perp_kernel, n_rows=n_rows),
        out_shape=jax.ShapeDtypeStruct((1, 8), jnp.float32),
    )(cnt_partials, scal)


@jax.jit
def kernel(z, embed, cluster_size):
    orig_shape = z.shape
    flat_z = z.reshape(-1, CODE_DIM)
    n_rows = flat_z.shape[0]
    codes, scal, active = _tc_stats(flat_z, embed, cluster_size)
    codes_flat = codes.reshape(-1)
    zeros_k = jnp.zeros((NUM_CODES,), jnp.float32)
    ones_b = jnp.ones((n_rows,), jnp.float32)
    zq, cnt_partials = _sc_gather_hist(embed, codes_flat, zeros_k, ones_b)
    scal = _tc_perplexity(cnt_partials, scal, n_rows)

    z_q_out = zq.reshape(orig_shape)
    codes_out = codes.reshape(orig_shape[:-1])
    return (z_q_out, codes_out,
            scal[0, 0], scal[0, 1], scal[0, 2], scal[0, 3], scal[0, 4],
            active[0, 0])
